# Initial kernel scaffold; baseline (speedup 1.0000x reference)
import jax
import jax.numpy as jnp
from jax.experimental import pallas as pl


def kernel(word_indices, emb_table, W1, b1, W2, b2, W3, b3):
    raise NotImplementedError("write your pallas kernel here")


# speedup vs baseline: 1.0324x; 1.0324x over previous
import functools

import jax
import jax.numpy as jnp
from jax import lax
from jax.experimental import pallas as pl
from jax.experimental.pallas import tpu as pltpu
from jax.experimental.pallas import tpu_sc as plsc

B = 4096
S = 20
E = 128
HID = 1024
NCLS = 2

NC = 2
NS = 16
NW = NC * NS
B_PER_W = B // NW
CHUNK = 4
N_CHUNKS = B_PER_W // CHUNK
IDX_PER_CHUNK = CHUNK * S
L = 16

def _gather_mean_body(idx_hbm, table_hbm, out_hbm, idx_v, rows_v, out_v, sem):
    wid = lax.axis_index("s") * NC + lax.axis_index("c")
    pltpu.sync_copy(idx_hbm.at[wid], idx_v)
    pltpu.async_copy(table_hbm.at[idx_v.at[0]], rows_v.at[0], sem)

    inv_s = jnp.float32(1.0 / S)

    def chunk_body(c, _):
        buf = lax.rem(c, 2)
        nxt = lax.rem(c + 1, 2)

        @pl.when(c + 1 < N_CHUNKS)
        def _prefetch():
            pltpu.async_copy(table_hbm.at[idx_v.at[c + 1]], rows_v.at[nxt], sem)

        pltpu.make_async_copy(
            table_hbm.at[idx_v.at[c]], rows_v.at[buf], sem
        ).wait()

        for r in range(CHUNK):
            for g in range(E // L):
                sl = pl.ds(g * L, L)
                acc = rows_v[buf, r * S, sl]
                for j in range(1, S):
                    acc = acc + rows_v[buf, r * S + j, sl]
                out_v[c * CHUNK + r, sl] = acc * inv_s
        return 0

    lax.fori_loop(0, N_CHUNKS, chunk_body, 0)
    pltpu.sync_copy(out_v, out_hbm.at[pl.ds(wid * B_PER_W, B_PER_W)])


@functools.cache
def _gather_mean():
    mesh = plsc.VectorSubcoreMesh(core_axis_name="c", subcore_axis_name="s")
    return pl.kernel(
        _gather_mean_body,
        mesh=mesh,
        out_type=jax.ShapeDtypeStruct((B, E), jnp.float32),
        scratch_types=[
            pltpu.VMEM((N_CHUNKS, IDX_PER_CHUNK), jnp.int32),
            pltpu.VMEM((2, IDX_PER_CHUNK, E), jnp.float32),
            pltpu.VMEM((B_PER_W, E), jnp.float32),
            pltpu.SemaphoreType.DMA,
        ],
    )


def _mlp_body(x_ref, w1_ref, b1_ref, w2_ref, b2_ref, w3_ref, b3_ref, o_ref):
    dn = (((1,), (1,)), ((), ()))
    x = x_ref[...]
    h = lax.dot_general(x, w1_ref[...], dn, preferred_element_type=jnp.float32,
                        precision=lax.Precision.HIGHEST)
    h = jnp.maximum(h + b1_ref[...], 0.0)
    h = lax.dot_general(h, w2_ref[...], dn, preferred_element_type=jnp.float32,
                        precision=lax.Precision.HIGHEST)
    h = jnp.maximum(h + b2_ref[...], 0.0)
    logits = lax.dot_general(h, w3_ref[...], dn,
                             preferred_element_type=jnp.float32,
                             precision=lax.Precision.HIGHEST)
    logits = logits + b3_ref[...]
    m = jnp.max(logits, axis=-1, keepdims=True)
    sh = logits - m
    lse = jnp.log(jnp.sum(jnp.exp(sh), axis=-1, keepdims=True))
    o_ref[...] = sh - lse


BB = 512


def _mlp(avg, W1, b1, W2, b2, W3, b3):
    grid = (B // BB,)
    return pl.pallas_call(
        _mlp_body,
        grid=grid,
        in_specs=[
            pl.BlockSpec((BB, E), lambda i: (i, 0)),
            pl.BlockSpec((HID, E), lambda i: (0, 0)),
            pl.BlockSpec((1, HID), lambda i: (0, 0)),
            pl.BlockSpec((HID, HID), lambda i: (0, 0)),
            pl.BlockSpec((1, HID), lambda i: (0, 0)),
            pl.BlockSpec((NCLS, HID), lambda i: (0, 0)),
            pl.BlockSpec((1, NCLS), lambda i: (0, 0)),
        ],
        out_specs=pl.BlockSpec((BB, NCLS), lambda i: (i, 0)),
        out_shape=jax.ShapeDtypeStruct((B, NCLS), jnp.float32),
    )(avg, W1, b1, W2, b2, W3, b3)


def kernel(word_indices, emb_table, W1, b1, W2, b2, W3, b3):
    idx = word_indices.reshape(NW, N_CHUNKS, IDX_PER_CHUNK).astype(jnp.int32)
    avg = _gather_mean()(idx, emb_table)
    return _mlp(avg, W1, b1.reshape(1, HID), W2, b2.reshape(1, HID),
                W3, b3.reshape(1, NCLS))


# speedup vs baseline: 1.9568x; 1.8954x over previous
import functools

import jax
import jax.numpy as jnp
from jax import lax
from jax.experimental import pallas as pl
from jax.experimental.pallas import tpu as pltpu
from jax.experimental.pallas import tpu_sc as plsc

B = 4096
S = 20
E = 128
HID = 1024
NCLS = 2

NC = 2
NS = 16
NW = NC * NS
B_PER_W = B // NW
CHUNK = 4
N_CHUNKS = B_PER_W // CHUNK
IDX_PER_CHUNK = CHUNK * S
L = 16

def _gather_mean_body(idx_hbm, table_hbm, out_hbm, idx_v, rows_v, out_v, sem):
    wid = lax.axis_index("s") * NC + lax.axis_index("c")
    pltpu.sync_copy(idx_hbm.at[wid], idx_v)
    pltpu.async_copy(table_hbm.at[idx_v.at[0]], rows_v.at[0], sem)

    inv_s = jnp.float32(1.0 / S)

    def chunk_body(c, _):
        buf = lax.rem(c, 2)
        nxt = lax.rem(c + 1, 2)

        @pl.when(c + 1 < N_CHUNKS)
        def _prefetch():
            pltpu.async_copy(table_hbm.at[idx_v.at[c + 1]], rows_v.at[nxt], sem)

        pltpu.make_async_copy(
            table_hbm.at[idx_v.at[c]], rows_v.at[buf], sem
        ).wait()

        for r in range(CHUNK):
            for g in range(E // L):
                sl = pl.ds(g * L, L)
                acc = rows_v[buf, r * S, sl]
                for j in range(1, S):
                    acc = acc + rows_v[buf, r * S + j, sl]
                out_v[c * CHUNK + r, sl] = acc * inv_s
        return 0

    lax.fori_loop(0, N_CHUNKS, chunk_body, 0)
    pltpu.sync_copy(out_v, out_hbm.at[pl.ds(wid * B_PER_W, B_PER_W)])


@functools.cache
def _gather_mean():
    mesh = plsc.VectorSubcoreMesh(core_axis_name="c", subcore_axis_name="s")
    return pl.kernel(
        _gather_mean_body,
        mesh=mesh,
        out_type=jax.ShapeDtypeStruct((B, E), jnp.float32),
        scratch_types=[
            pltpu.VMEM((N_CHUNKS, IDX_PER_CHUNK), jnp.int32),
            pltpu.VMEM((2, IDX_PER_CHUNK, E), jnp.float32),
            pltpu.VMEM((B_PER_W, E), jnp.float32),
            pltpu.SemaphoreType.DMA,
        ],
    )


def _mlp_body(x_ref, w1_ref, b1_ref, w2_ref, b2_ref, w3_ref, b3_ref, o_ref):
    dn = (((1,), (1,)), ((), ()))
    x = x_ref[...]
    h = lax.dot_general(x, w1_ref[...], dn, preferred_element_type=jnp.float32,
                        precision=lax.Precision.DEFAULT)
    h = jnp.maximum(h + b1_ref[...], 0.0)
    h = lax.dot_general(h, w2_ref[...], dn, preferred_element_type=jnp.float32,
                        precision=lax.Precision.DEFAULT)
    h = jnp.maximum(h + b2_ref[...], 0.0)
    logits = lax.dot_general(h, w3_ref[...], dn,
                             preferred_element_type=jnp.float32,
                             precision=lax.Precision.DEFAULT)
    logits = logits + b3_ref[...]
    m = jnp.max(logits, axis=-1, keepdims=True)
    sh = logits - m
    lse = jnp.log(jnp.sum(jnp.exp(sh), axis=-1, keepdims=True))
    o_ref[...] = sh - lse


BB = 512


def _mlp(avg, W1, b1, W2, b2, W3, b3):
    grid = (B // BB,)
    return pl.pallas_call(
        _mlp_body,
        grid=grid,
        in_specs=[
            pl.BlockSpec((BB, E), lambda i: (i, 0)),
            pl.BlockSpec((HID, E), lambda i: (0, 0)),
            pl.BlockSpec((1, HID), lambda i: (0, 0)),
            pl.BlockSpec((HID, HID), lambda i: (0, 0)),
            pl.BlockSpec((1, HID), lambda i: (0, 0)),
            pl.BlockSpec((NCLS, HID), lambda i: (0, 0)),
            pl.BlockSpec((1, NCLS), lambda i: (0, 0)),
        ],
        out_specs=pl.BlockSpec((BB, NCLS), lambda i: (i, 0)),
        out_shape=jax.ShapeDtypeStruct((B, NCLS), jnp.float32),
    )(avg, W1, b1, W2, b2, W3, b3)


def kernel(word_indices, emb_table, W1, b1, W2, b2, W3, b3):
    idx = word_indices.reshape(NW, N_CHUNKS, IDX_PER_CHUNK).astype(jnp.int32)
    avg = _gather_mean()(idx, emb_table)
    return _mlp(avg, W1, b1.reshape(1, HID), W2, b2.reshape(1, HID),
                W3, b3.reshape(1, NCLS))
